# baseline (speedup 1.0000x reference)
import functools

import jax
import jax.numpy as jnp
from jax import lax
from jax.experimental import pallas as pl
from jax.experimental.pallas import tpu as pltpu
from jax.experimental.pallas import tpu_sc as plsc

NUM_POS = 4096
NUM_DIM = 1024
BATCH = 4

_NC = 2
_NS = 16
_NW = _NC * _NS
_ROWS_PER_W = NUM_POS // _NW

_mesh = plsc.VectorSubcoreMesh(core_axis_name="c", subcore_axis_name="s")


@functools.partial(
    pl.kernel,
    mesh=_mesh,
    out_type=jax.ShapeDtypeStruct((BATCH, NUM_POS, NUM_DIM), jnp.float32),
    scratch_types=[pltpu.SemaphoreType.DMA],
)
def _broadcast_table(w_hbm, out_hbm, wsem):
    wid = lax.axis_index("s") * _NC + lax.axis_index("c")
    base = wid * _ROWS_PER_W
    writes = [
        pltpu.async_copy(
            w_hbm.at[pl.ds(base, _ROWS_PER_W)],
            out_hbm.at[b, pl.ds(base, _ROWS_PER_W)],
            wsem,
        )
        for b in range(BATCH)
    ]
    for wr in writes:
        wr.wait()


def kernel(x, w):
    del x
    return _broadcast_table(w)


# speedup vs baseline: 43.2021x; 43.2021x over previous
import functools

import jax
import jax.numpy as jnp
from jax import lax
from jax.experimental import pallas as pl
from jax.experimental.pallas import tpu as pltpu
from jax.experimental.pallas import tpu_sc as plsc

NUM_POS = 4096
NUM_DIM = 1024
BATCH = 4

_NC = 2
_NS = 16
_NW = _NC * _NS
_ROWS_PER_W = NUM_POS // _NW
_CHUNK = 32
_NCH = _ROWS_PER_W // _CHUNK
_NBUF = 3

_mesh = plsc.VectorSubcoreMesh(core_axis_name="c", subcore_axis_name="s")


@functools.partial(
    pl.kernel,
    mesh=_mesh,
    out_type=jax.ShapeDtypeStruct((BATCH, NUM_POS, NUM_DIM), jnp.float32),
    scratch_types=(
        [pltpu.VMEM((_CHUNK, NUM_DIM), jnp.float32) for _ in range(_NBUF)]
        + [
            pltpu.VMEM_SHARED((_NS * _CHUNK, NUM_DIM), jnp.float32),
            pltpu.SemaphoreType.DMA,
            pltpu.SemaphoreType.DMA,
            pltpu.SemaphoreType.DMA,
            pltpu.SemaphoreType.DMA,
        ]
    ),
)
def _broadcast_table(w_hbm, out_hbm, buf0, buf1, buf2, shared,
                     rsem, wsem, lsem, ssem):
    bufs = (buf0, buf1, buf2)
    sid = lax.axis_index("s")
    wid = sid * _NC + lax.axis_index("c")
    base = wid * _ROWS_PER_W
    sbase = sid * _CHUNK

    reads = {}
    for c in range(_NBUF):
        reads[c] = pltpu.async_copy(
            w_hbm.at[pl.ds(base + c * _CHUNK, _CHUNK)], bufs[c % _NBUF], rsem)

    twrites = []
    swrites = {}
    for c in range(_NCH):
        reads[c].wait()
        for b in range(2):
            twrites.append(pltpu.async_copy(
                bufs[c % _NBUF],
                out_hbm.at[b, pl.ds(base + c * _CHUNK, _CHUNK)],
                wsem,
            ))
        if c >= 1:
            for h in swrites.pop(c - 1):
                h.wait()
        slot = sbase
        pltpu.async_copy(
            bufs[c % _NBUF], shared.at[pl.ds(slot, _CHUNK)], lsem).wait()
        swrites[c] = [
            pltpu.async_copy(
                shared.at[pl.ds(slot, _CHUNK)],
                out_hbm.at[b, pl.ds(base + c * _CHUNK, _CHUNK)],
                ssem,
            )
            for b in range(2, BATCH)
        ]
        nxt = c + _NBUF
        if nxt < _NCH:
            for _ in range(2):
                twrites.pop(0).wait()
            reads[nxt] = pltpu.async_copy(
                w_hbm.at[pl.ds(base + nxt * _CHUNK, _CHUNK)],
                bufs[nxt % _NBUF], rsem)
    for wr in twrites:
        wr.wait()
    for hs in swrites.values():
        for h in hs:
            h.wait()


def kernel(x, w):
    del x
    return _broadcast_table(w)


# speedup vs baseline: 45.1677x; 1.0455x over previous
import functools

import jax
import jax.numpy as jnp
from jax import lax
from jax.experimental import pallas as pl
from jax.experimental.pallas import tpu as pltpu
from jax.experimental.pallas import tpu_sc as plsc

NUM_POS = 4096
NUM_DIM = 1024
BATCH = 4

_NC = 2
_NS = 16
_NW = _NC * _NS
_ROWS_PER_W = NUM_POS // _NW
_CHUNK = 32
_NCH = _ROWS_PER_W // _CHUNK
_NBUF = 3

_mesh = plsc.VectorSubcoreMesh(core_axis_name="c", subcore_axis_name="s")


@functools.partial(
    pl.kernel,
    mesh=_mesh,
    out_type=jax.ShapeDtypeStruct((BATCH, NUM_POS, NUM_DIM), jnp.float32),
    scratch_types=(
        [pltpu.VMEM((_CHUNK, NUM_DIM), jnp.float32) for _ in range(_NBUF)]
        + [pltpu.SemaphoreType.DMA, pltpu.SemaphoreType.DMA]
    ),
)
def _broadcast_table(w_hbm, out_hbm, buf0, buf1, buf2, rsem, wsem):
    bufs = (buf0, buf1, buf2)
    wid = lax.axis_index("s") * _NC + lax.axis_index("c")
    base = wid * _ROWS_PER_W

    reads = {}
    for c in range(_NBUF):
        reads[c] = pltpu.async_copy(
            w_hbm.at[pl.ds(base + c * _CHUNK, _CHUNK)], bufs[c % _NBUF], rsem)

    writes = []
    for c in range(_NCH):
        reads[c].wait()
        for b in range(BATCH):
            writes.append(pltpu.async_copy(
                bufs[c % _NBUF],
                out_hbm.at[b, pl.ds(base + c * _CHUNK, _CHUNK)],
                wsem,
            ))
        nxt = c + _NBUF
        if nxt < _NCH:
            for _ in range(BATCH):
                writes.pop(0).wait()
            reads[nxt] = pltpu.async_copy(
                w_hbm.at[pl.ds(base + nxt * _CHUNK, _CHUNK)],
                bufs[nxt % _NBUF], rsem)
    for wr in writes:
        wr.wait()


def kernel(x, w):
    del x
    return _broadcast_table(w)
